# baseline (speedup 1.0000x reference)
import jax
import jax.numpy as jnp
from jax.experimental import pallas as pl
from jax.experimental.pallas import tpu as pltpu

_NBINS = 11


def _drmm_step(qn_ref, dn_ref, q_ref, qid_ref, did_ref, w1_ref, wg_ref, b1_ref,
               out_ref, facc, lacc, hb16):
    b = pl.program_id(0)
    nb = pl.num_programs(0)
    qnb = qn_ref[0]
    dnb = dn_ref[0]
    q = q_ref[0]
    wq = (qid_ref[0] != 0).astype(jnp.float32)
    wd = (did_ref[0] != 0).astype(jnp.float32)

    sim = jax.lax.dot_general(qnb, dnb,
                              (((1,), (1,)), ((), ())),
                              preferred_element_type=jnp.float32)
    scaled = ((sim + 1.00001) / 2.0) * float(_NBINS - 1)
    idx = jnp.clip(scaled.astype(jnp.int32), 0, _NBINS - 1)

    idx1 = idx | 1
    val = 1.0 + 4095.0 * (idx & 1).astype(jnp.float32)
    cols = []
    for g in range(5):
        mg = jnp.where(idx1 == 2 * g + 1, val, 0.0)
        rg = jax.lax.dot_general(mg, wd, (((1,), (1,)), ((), ())),
                                 preferred_element_type=jnp.float32)
        c_odd = (rg * (1.0 / 4096.0)).astype(jnp.int32).astype(jnp.float32)
        c_even = rg - 4096.0 * c_odd
        cols.extend([c_even, c_odd])
    ndocs = jnp.sum(wd)
    hist10 = jnp.concatenate(cols, axis=1)
    c10 = ndocs - jnp.sum(hist10, axis=1, keepdims=True)
    hist = jnp.concatenate([hist10, c10], axis=1)
    h = jnp.log(hist * wq + 1e-5)

    hb16[...] = h.astype(jnp.bfloat16)
    hb = hb16[...].astype(jnp.float32)
    facc[pl.ds(b, 1), :] = jnp.sum(hb * w1_ref[...], keepdims=True)
    lacc[pl.ds(b, 1), :] = jnp.sum(q * wg_ref[...], keepdims=True)

    @pl.when(b == nb - 1)
    def _():
        logits = lacc[...]
        m = jnp.max(logits)
        e = jnp.exp(logits - m)
        s = jnp.sum(e)
        t = jnp.sum(e / s)
        out_ref[...] = (facc[...] + b1_ref[...]) * t


def kernel(q_embeddings, d_embeddings, q_embeddings_ids, d_embeddings_ids,
           W1, b1, Wg):
    B, Q, E = q_embeddings.shape
    D = d_embeddings.shape[1]
    qids3 = q_embeddings_ids.reshape(B, Q, 1)
    dids3 = d_embeddings_ids.reshape(B, 1, D)
    w1r = jax.lax.reduce_precision(W1.reshape(Q, _NBINS),
                                   exponent_bits=8, mantissa_bits=7)
    wgr = Wg.reshape(Q, E)
    b1r = b1.reshape(1, 1)
    qnb = (q_embeddings / (jnp.linalg.norm(q_embeddings, axis=-1, keepdims=True)
                           + 1e-13)).astype(jnp.bfloat16)
    dnb = (d_embeddings / (jnp.linalg.norm(d_embeddings, axis=-1, keepdims=True)
                           + 1e-13)).astype(jnp.bfloat16)

    out = pl.pallas_call(
        _drmm_step,
        grid=(B,),
        in_specs=[
            pl.BlockSpec((1, Q, E), lambda b: (b, 0, 0)),
            pl.BlockSpec((1, D, E), lambda b: (b, 0, 0)),
            pl.BlockSpec((1, Q, E), lambda b: (b, 0, 0)),
            pl.BlockSpec((1, Q, 1), lambda b: (b, 0, 0)),
            pl.BlockSpec((1, 1, D), lambda b: (b, 0, 0)),
            pl.BlockSpec((Q, _NBINS), lambda b: (0, 0)),
            pl.BlockSpec((Q, E), lambda b: (0, 0)),
            pl.BlockSpec((1, 1), lambda b: (0, 0)),
        ],
        out_specs=pl.BlockSpec((B, 1), lambda b: (0, 0)),
        out_shape=jax.ShapeDtypeStruct((B, 1), jnp.float32),
        scratch_shapes=[
            pltpu.VMEM((B, 1), jnp.float32),
            pltpu.VMEM((B, 1), jnp.float32),
            pltpu.VMEM((Q, _NBINS), jnp.bfloat16),
        ],
    )(qnb, dnb, q_embeddings, qids3, dids3, w1r, wgr, b1r)
    return out.reshape(B)


# speedup vs baseline: 1.0280x; 1.0280x over previous
import jax
import jax.numpy as jnp
from jax.experimental import pallas as pl
from jax.experimental.pallas import tpu as pltpu

_NBINS = 11


def _drmm_step(qn_ref, dn_ref, q_ref, qid_ref, did_ref, w1_ref, wg_ref, b1_ref,
               out_ref, facc, lacc, hb16):
    b = pl.program_id(0)
    nb = pl.num_programs(0)
    qnb = qn_ref[0]
    dnb = dn_ref[0]
    q = q_ref[0]
    wq = (qid_ref[0] != 0).astype(jnp.float32)
    wd = (did_ref[0] != 0).astype(jnp.float32)

    sim = jax.lax.dot_general(qnb, dnb,
                              (((1,), (1,)), ((), ())),
                              preferred_element_type=jnp.float32)
    scaled = ((sim + 1.00001) / 2.0) * float(_NBINS - 1)
    idx = scaled.astype(jnp.int32)

    idx1 = idx | 1
    val = 1.0 + 4095.0 * (idx & 1).astype(jnp.float32)
    cols = []
    for g in range(5):
        mg = jnp.where(idx1 == 2 * g + 1, val, 0.0)
        rg = jax.lax.dot_general(mg, wd, (((1,), (1,)), ((), ())),
                                 preferred_element_type=jnp.float32)
        c_odd = (rg * (1.0 / 4096.0)).astype(jnp.int32).astype(jnp.float32)
        c_even = rg - 4096.0 * c_odd
        cols.extend([c_even, c_odd])
    ndocs = jnp.sum(wd)
    hist10 = jnp.concatenate(cols, axis=1)
    c10 = ndocs - jnp.sum(hist10, axis=1, keepdims=True)
    hist = jnp.concatenate([hist10, c10], axis=1)
    h = jnp.log(hist * wq + 1e-5)

    hb16[...] = h.astype(jnp.bfloat16)
    hb = hb16[...].astype(jnp.float32)
    facc[pl.ds(b, 1), :] = jnp.sum(hb * w1_ref[...], keepdims=True)
    lacc[pl.ds(b, 1), :] = jnp.sum(q * wg_ref[...], keepdims=True)

    @pl.when(b == nb - 1)
    def _():
        logits = lacc[...]
        m = jnp.max(logits)
        e = jnp.exp(logits - m)
        s = jnp.sum(e)
        t = jnp.sum(e / s)
        out_ref[...] = (facc[...] + b1_ref[...]) * t


def kernel(q_embeddings, d_embeddings, q_embeddings_ids, d_embeddings_ids,
           W1, b1, Wg):
    B, Q, E = q_embeddings.shape
    D = d_embeddings.shape[1]
    qids3 = q_embeddings_ids.reshape(B, Q, 1)
    dids3 = d_embeddings_ids.reshape(B, 1, D)
    w1r = jax.lax.reduce_precision(W1.reshape(Q, _NBINS),
                                   exponent_bits=8, mantissa_bits=7)
    wgr = Wg.reshape(Q, E)
    b1r = b1.reshape(1, 1)
    qnb = (q_embeddings / (jnp.linalg.norm(q_embeddings, axis=-1, keepdims=True)
                           + 1e-13)).astype(jnp.bfloat16)
    dnb = (d_embeddings / (jnp.linalg.norm(d_embeddings, axis=-1, keepdims=True)
                           + 1e-13)).astype(jnp.bfloat16)

    out = pl.pallas_call(
        _drmm_step,
        grid=(B,),
        in_specs=[
            pl.BlockSpec((1, Q, E), lambda b: (b, 0, 0)),
            pl.BlockSpec((1, D, E), lambda b: (b, 0, 0)),
            pl.BlockSpec((1, Q, E), lambda b: (b, 0, 0)),
            pl.BlockSpec((1, Q, 1), lambda b: (b, 0, 0)),
            pl.BlockSpec((1, 1, D), lambda b: (b, 0, 0)),
            pl.BlockSpec((Q, _NBINS), lambda b: (0, 0)),
            pl.BlockSpec((Q, E), lambda b: (0, 0)),
            pl.BlockSpec((1, 1), lambda b: (0, 0)),
        ],
        out_specs=pl.BlockSpec((B, 1), lambda b: (0, 0)),
        out_shape=jax.ShapeDtypeStruct((B, 1), jnp.float32),
        scratch_shapes=[
            pltpu.VMEM((B, 1), jnp.float32),
            pltpu.VMEM((B, 1), jnp.float32),
            pltpu.VMEM((Q, _NBINS), jnp.bfloat16),
        ],
    )(qnb, dnb, q_embeddings, qids3, dids3, w1r, wgr, b1r)
    return out.reshape(B)
